# baseline (speedup 1.0000x reference)
import jax
import jax.numpy as jnp
from jax.experimental import pallas as pl

B = 128
H = 2048
I = 1024
K = 8
LOCAL = 8


def _mlp1_body(x_ref, w1_ref, eid_ref, sc_ref, act_ref):
    e = pl.program_id(0)
    f32 = jnp.float32
    bf16 = jnp.bfloat16
    xb = x_ref[...].astype(bf16)
    w1 = w1_ref[0].reshape(H, 2 * I).astype(bf16)
    h1 = jnp.dot(xb, w1, preferred_element_type=f32)
    gate = h1[:, :I]
    up = h1[:, I:]
    w = jnp.sum(jnp.where(eid_ref[...] == e, sc_ref[...], 0.0), axis=1)
    a = gate * jax.nn.sigmoid(gate) * up * w[:, None]
    act_ref[0] = a.astype(bf16)


def _mlp2_body(act_ref, w2_ref, out_ref):
    e = pl.program_id(0)

    @pl.when(e == 0)
    def _():
        out_ref[...] = jnp.zeros_like(out_ref)

    out_ref[...] += jnp.dot(act_ref[0], w2_ref[0].astype(jnp.bfloat16),
                            preferred_element_type=jnp.float32)


def kernel(x, expert_ids, smooth_scales, expert_scales, x_active_mask,
           gmm1_weight, gmm2_weight):
    del smooth_scales
    eids = expert_ids.astype(jnp.int32)
    sc = expert_scales * x_active_mask[:, None].astype(jnp.float32)
    w1 = gmm1_weight.reshape(LOCAL, H, 2, I)

    act = pl.pallas_call(
        _mlp1_body,
        grid=(LOCAL,),
        in_specs=[
            pl.BlockSpec((B, H), lambda e: (0, 0)),
            pl.BlockSpec((1, H, 2, I), lambda e: (e, 0, 0, 0)),
            pl.BlockSpec((B, K), lambda e: (0, 0)),
            pl.BlockSpec((B, K), lambda e: (0, 0)),
        ],
        out_specs=pl.BlockSpec((1, B, I), lambda e: (e, 0, 0)),
        out_shape=jax.ShapeDtypeStruct((LOCAL, B, I), jnp.bfloat16),
    )(x, w1, eids, sc)

    out = pl.pallas_call(
        _mlp2_body,
        grid=(LOCAL,),
        in_specs=[
            pl.BlockSpec((1, B, I), lambda e: (e, 0, 0)),
            pl.BlockSpec((1, I, H), lambda e: (e, 0, 0)),
        ],
        out_specs=pl.BlockSpec((B, H), lambda e: (0, 0)),
        out_shape=jax.ShapeDtypeStruct((B, H), jnp.float32),
    )(act, gmm2_weight)
    return out


# speedup vs baseline: 1.0217x; 1.0217x over previous
import jax
import jax.numpy as jnp
from jax.experimental import pallas as pl

B = 128
H = 2048
I = 1024
K = 8
LOCAL = 8
NSPLIT = 2
IS = I // NSPLIT


def _moe_body(x_ref, w1_ref, w2_ref, eid_ref, sc_ref, out_ref):
    e = pl.program_id(0)
    j = pl.program_id(1)
    f32 = jnp.float32
    bf16 = jnp.bfloat16
    xb = x_ref[...].astype(bf16)
    w1 = w1_ref[0].reshape(H, 2 * IS).astype(bf16)
    h1 = jnp.dot(xb, w1, preferred_element_type=f32)
    gate = h1[:, :IS]
    up = h1[:, IS:]
    w = jnp.sum(jnp.where(eid_ref[...] == e, sc_ref[...], 0.0), axis=1)
    a = gate * jax.nn.sigmoid(gate) * up * w[:, None]

    @pl.when(jnp.logical_and(e == 0, j == 0))
    def _():
        out_ref[...] = jnp.zeros_like(out_ref)

    out_ref[...] += jnp.dot(a.astype(bf16), w2_ref[0].astype(bf16),
                            preferred_element_type=f32)


def kernel(x, expert_ids, smooth_scales, expert_scales, x_active_mask,
           gmm1_weight, gmm2_weight):
    del smooth_scales
    eids = expert_ids.astype(jnp.int32)
    sc = expert_scales * x_active_mask[:, None].astype(jnp.float32)
    w1 = gmm1_weight.reshape(LOCAL, H, 2, I)

    out = pl.pallas_call(
        _moe_body,
        grid=(LOCAL, NSPLIT),
        in_specs=[
            pl.BlockSpec((B, H), lambda e, j: (0, 0)),
            pl.BlockSpec((1, H, 2, IS), lambda e, j: (e, 0, 0, j)),
            pl.BlockSpec((1, IS, H), lambda e, j: (e * NSPLIT + j, 0, 0)),
            pl.BlockSpec((B, K), lambda e, j: (0, 0)),
            pl.BlockSpec((B, K), lambda e, j: (0, 0)),
        ],
        out_specs=pl.BlockSpec((B, H), lambda e, j: (0, 0)),
        out_shape=jax.ShapeDtypeStruct((B, H), jnp.float32),
    )(x, w1, gmm2_weight.reshape(LOCAL * NSPLIT, IS, H), eids, sc)
    return out
